# baseline (speedup 1.0000x reference)
import jax
import jax.numpy as jnp
from jax.experimental import pallas as pl
from jax.experimental.pallas import tpu as pltpu

T = 2048
D = 1024
E = 8
DFF = 512
HC = 256
TB = 256
NTB = T // TB


def _expert_pass(wgsrc, wusrc, wdsrc, e, xbf_ref, wfull_ref, out_ref,
                 wgu_ref, wd_ref, shared):
    wgu_ref[0 * HC:1 * HC, :] = wgsrc[0:HC, :].astype(jnp.bfloat16)
    wgu_ref[1 * HC:2 * HC, :] = wusrc[0:HC, :].astype(jnp.bfloat16)
    wgu_ref[2 * HC:3 * HC, :] = wgsrc[HC:2 * HC, :].astype(jnp.bfloat16)
    wgu_ref[3 * HC:4 * HC, :] = wusrc[HC:2 * HC, :].astype(jnp.bfloat16)
    wd_ref[:, 0:HC] = wdsrc[:, 0:HC].astype(jnp.bfloat16)
    wd_ref[:, HC:2 * HC] = wdsrc[:, HC:2 * HC].astype(jnp.bfloat16)

    lane = jax.lax.broadcasted_iota(jnp.int32, (TB, E), 1)
    dn = (((1,), (1,)), ((), ()))
    for tb in range(NTB):
        rows = pl.ds(tb * TB, TB)
        xb = xbf_ref[rows, :]
        if shared:
            w = 1.0
        else:
            w = jnp.sum(jnp.where(lane == e, wfull_ref[rows, :], 0.0),
                        axis=-1, keepdims=True)
        g0 = jax.lax.dot_general(xb, wgu_ref[0 * HC:1 * HC, :], dn,
                                 preferred_element_type=jnp.float32)
        u0 = jax.lax.dot_general(xb, wgu_ref[1 * HC:2 * HC, :], dn,
                                 preferred_element_type=jnp.float32)
        h0 = ((g0 * (1.0 / (1.0 + jnp.exp(-g0)))) * u0 * w).astype(jnp.bfloat16)
        o0 = jax.lax.dot_general(h0, wd_ref[:, 0:HC], dn,
                                 preferred_element_type=jnp.float32)
        g1 = jax.lax.dot_general(xb, wgu_ref[2 * HC:3 * HC, :], dn,
                                 preferred_element_type=jnp.float32)
        u1 = jax.lax.dot_general(xb, wgu_ref[3 * HC:4 * HC, :], dn,
                                 preferred_element_type=jnp.float32)
        h1 = ((g1 * (1.0 / (1.0 + jnp.exp(-g1)))) * u1 * w).astype(jnp.bfloat16)
        o1 = jax.lax.dot_general(h1, wd_ref[:, HC:2 * HC], dn,
                                 preferred_element_type=jnp.float32)
        out_ref[rows, :] += o0 + o1


def _moe_body(x_ref, gate_ref, w1g_ref, w1u_ref, w2_ref, swg_ref, swu_ref,
              swd_ref, out_ref, xbf_ref, wfull_ref, wgu_ref, wd_ref):
    e = pl.program_id(0)

    @pl.when(e == 0)
    def _router():
        x = x_ref[...]
        xbf_ref[...] = x.astype(jnp.bfloat16)
        logits = jax.lax.dot_general(
            x, gate_ref[...], (((1,), (1,)), ((), ())),
            preferred_element_type=jnp.float32)
        m = jnp.max(logits, axis=-1, keepdims=True)
        ex = jnp.exp(logits - m)
        probs = ex / jnp.sum(ex, axis=-1, keepdims=True)
        lane = jax.lax.broadcasted_iota(jnp.int32, (T, E), 1)
        v1 = jnp.max(probs, axis=-1, keepdims=True)
        i1 = jnp.min(jnp.where(probs == v1, lane, E), axis=-1, keepdims=True)
        m1 = lane == i1
        probs2 = jnp.where(m1, -1.0, probs)
        v2 = jnp.max(probs2, axis=-1, keepdims=True)
        i2 = jnp.min(jnp.where(probs2 == v2, lane, E), axis=-1, keepdims=True)
        m2 = lane == i2
        denom = v1 + v2
        wfull_ref[...] = (jnp.where(m1, v1, 0.0) + jnp.where(m2, v2, 0.0)) / denom
        out_ref[...] = jnp.zeros((T, D), jnp.float32)

    @pl.when(e < E)
    def _routed():
        _expert_pass(w1g_ref[0], w1u_ref[0], w2_ref[0], e, xbf_ref,
                     wfull_ref, out_ref, wgu_ref, wd_ref, shared=False)

    @pl.when(e == E)
    def _shared():
        _expert_pass(swg_ref[...], swu_ref[...], swd_ref[...], e, xbf_ref,
                     wfull_ref, out_ref, wgu_ref, wd_ref, shared=True)


@jax.jit
def kernel(hidden_states, gate_w, w1_gate, w1_up, w2, sw_gate, sw_up, sw_down):
    grid = (E + 1,)
    out = pl.pallas_call(
        _moe_body,
        grid=grid,
        in_specs=[
            pl.BlockSpec((T, D), lambda e: (0, 0)),
            pl.BlockSpec((E, D), lambda e: (0, 0)),
            pl.BlockSpec((1, DFF, D), lambda e: (jnp.minimum(e, E - 1), 0, 0)),
            pl.BlockSpec((1, DFF, D), lambda e: (jnp.minimum(e, E - 1), 0, 0)),
            pl.BlockSpec((1, D, DFF), lambda e: (jnp.minimum(e, E - 1), 0, 0)),
            pl.BlockSpec((DFF, D), lambda e: (0, 0)),
            pl.BlockSpec((DFF, D), lambda e: (0, 0)),
            pl.BlockSpec((D, DFF), lambda e: (0, 0)),
        ],
        out_specs=pl.BlockSpec((T, D), lambda e: (0, 0)),
        out_shape=jax.ShapeDtypeStruct((T, D), jnp.float32),
        scratch_shapes=[
            pltpu.VMEM((T, D), jnp.bfloat16),
            pltpu.VMEM((T, E), jnp.float32),
            pltpu.VMEM((2 * DFF, D), jnp.bfloat16),
            pltpu.VMEM((D, DFF), jnp.bfloat16),
        ],
        compiler_params=pltpu.CompilerParams(
            dimension_semantics=("arbitrary",)),
    )(hidden_states, gate_w, w1_gate, w1_up, w2, sw_gate, sw_up, sw_down)
    return out


# speedup vs baseline: 1.1231x; 1.1231x over previous
import jax
import jax.numpy as jnp
from jax.experimental import pallas as pl
from jax.experimental.pallas import tpu as pltpu

T = 2048
D = 1024
E = 8
DFF = 512
HC = 256
TB = 512
NTB = T // TB


def _expert_pass(wgsrc, wusrc, wdsrc, e, xbf_ref, wfull_ref, out_ref,
                 wgu_ref, wd_ref, hs_ref, shared):
    wgu_ref[0 * HC:1 * HC, :] = wgsrc[0:HC, :].astype(jnp.bfloat16)
    wgu_ref[1 * HC:2 * HC, :] = wusrc[0:HC, :].astype(jnp.bfloat16)
    wgu_ref[2 * HC:3 * HC, :] = wgsrc[HC:2 * HC, :].astype(jnp.bfloat16)
    wgu_ref[3 * HC:4 * HC, :] = wusrc[HC:2 * HC, :].astype(jnp.bfloat16)
    wd_ref[:, 0:HC] = wdsrc[:, 0:HC].astype(jnp.bfloat16)
    wd_ref[:, HC:2 * HC] = wdsrc[:, HC:2 * HC].astype(jnp.bfloat16)

    lane = jax.lax.broadcasted_iota(jnp.int32, (TB, E), 1)
    dn = (((1,), (1,)), ((), ()))
    for tb in range(NTB):
        rows = pl.ds(tb * TB, TB)
        xb = xbf_ref[rows, :]
        if shared:
            w = 1.0
        else:
            w = jnp.sum(jnp.where(lane == e, wfull_ref[rows, :], 0.0),
                        axis=-1, keepdims=True)
        g0 = jax.lax.dot_general(xb, wgu_ref[0 * HC:1 * HC, :], dn,
                                 preferred_element_type=jnp.float32)
        u0 = jax.lax.dot_general(xb, wgu_ref[1 * HC:2 * HC, :], dn,
                                 preferred_element_type=jnp.float32)
        hs_ref[:, 0:HC] = ((g0 * (1.0 / (1.0 + jnp.exp(-g0)))) * u0 * w
                           ).astype(jnp.bfloat16)
        g1 = jax.lax.dot_general(xb, wgu_ref[2 * HC:3 * HC, :], dn,
                                 preferred_element_type=jnp.float32)
        u1 = jax.lax.dot_general(xb, wgu_ref[3 * HC:4 * HC, :], dn,
                                 preferred_element_type=jnp.float32)
        hs_ref[:, HC:2 * HC] = ((g1 * (1.0 / (1.0 + jnp.exp(-g1)))) * u1 * w
                                ).astype(jnp.bfloat16)
        o = jax.lax.dot_general(hs_ref[...], wd_ref[...], dn,
                                preferred_element_type=jnp.float32)
        out_ref[rows, :] += o


def _moe_body(x_ref, gate_ref, w1g_ref, w1u_ref, w2_ref, swg_ref, swu_ref,
              swd_ref, out_ref, xbf_ref, wfull_ref, wgu_ref, wd_ref, hs_ref):
    e = pl.program_id(0)

    @pl.when(e == 0)
    def _router():
        x = x_ref[...]
        xbf_ref[...] = x.astype(jnp.bfloat16)
        logits = jax.lax.dot_general(
            x, gate_ref[...], (((1,), (1,)), ((), ())),
            preferred_element_type=jnp.float32)
        m = jnp.max(logits, axis=-1, keepdims=True)
        ex = jnp.exp(logits - m)
        probs = ex / jnp.sum(ex, axis=-1, keepdims=True)
        lane = jax.lax.broadcasted_iota(jnp.int32, (T, E), 1)
        v1 = jnp.max(probs, axis=-1, keepdims=True)
        i1 = jnp.min(jnp.where(probs == v1, lane, E), axis=-1, keepdims=True)
        m1 = lane == i1
        probs2 = jnp.where(m1, -1.0, probs)
        v2 = jnp.max(probs2, axis=-1, keepdims=True)
        i2 = jnp.min(jnp.where(probs2 == v2, lane, E), axis=-1, keepdims=True)
        m2 = lane == i2
        denom = v1 + v2
        wfull_ref[...] = (jnp.where(m1, v1, 0.0) + jnp.where(m2, v2, 0.0)) / denom
        out_ref[...] = jnp.zeros((T, D), jnp.float32)

    @pl.when(e < E)
    def _routed():
        _expert_pass(w1g_ref[0], w1u_ref[0], w2_ref[0], e, xbf_ref,
                     wfull_ref, out_ref, wgu_ref, wd_ref, hs_ref, shared=False)

    @pl.when(e == E)
    def _shared():
        _expert_pass(swg_ref[...], swu_ref[...], swd_ref[...], e, xbf_ref,
                     wfull_ref, out_ref, wgu_ref, wd_ref, hs_ref, shared=True)


@jax.jit
def kernel(hidden_states, gate_w, w1_gate, w1_up, w2, sw_gate, sw_up, sw_down):
    grid = (E + 1,)
    out = pl.pallas_call(
        _moe_body,
        grid=grid,
        in_specs=[
            pl.BlockSpec((T, D), lambda e: (0, 0)),
            pl.BlockSpec((E, D), lambda e: (0, 0)),
            pl.BlockSpec((1, DFF, D), lambda e: (jnp.minimum(e, E - 1), 0, 0)),
            pl.BlockSpec((1, DFF, D), lambda e: (jnp.minimum(e, E - 1), 0, 0)),
            pl.BlockSpec((1, D, DFF), lambda e: (jnp.minimum(e, E - 1), 0, 0)),
            pl.BlockSpec((DFF, D), lambda e: (0, 0)),
            pl.BlockSpec((DFF, D), lambda e: (0, 0)),
            pl.BlockSpec((D, DFF), lambda e: (0, 0)),
        ],
        out_specs=pl.BlockSpec((T, D), lambda e: (0, 0)),
        out_shape=jax.ShapeDtypeStruct((T, D), jnp.float32),
        scratch_shapes=[
            pltpu.VMEM((T, D), jnp.bfloat16),
            pltpu.VMEM((T, E), jnp.float32),
            pltpu.VMEM((2 * DFF, D), jnp.bfloat16),
            pltpu.VMEM((D, DFF), jnp.bfloat16),
            pltpu.VMEM((TB, DFF), jnp.bfloat16),
        ],
        compiler_params=pltpu.CompilerParams(
            dimension_semantics=("arbitrary",)),
    )(hidden_states, gate_w, w1_gate, w1_up, w2, sw_gate, sw_up, sw_down)
    return out


# speedup vs baseline: 1.1268x; 1.0032x over previous
import jax
import jax.numpy as jnp
from jax.experimental import pallas as pl
from jax.experimental.pallas import tpu as pltpu

T = 2048
D = 1024
E = 8
DFF = 512
HC = 256
TB = 512
NTB = T // TB


def _expert_pass(wgsrc, wusrc, wdsrc, e, xbf_ref, wfull_ref, out_ref,
                 wgu_ref, wd_ref, hs_ref, shared):
    wgu_ref[0 * HC:1 * HC, :] = wgsrc[0:HC, :].astype(jnp.bfloat16)
    wgu_ref[1 * HC:2 * HC, :] = wusrc[0:HC, :].astype(jnp.bfloat16)
    wgu_ref[2 * HC:3 * HC, :] = wgsrc[HC:2 * HC, :].astype(jnp.bfloat16)
    wgu_ref[3 * HC:4 * HC, :] = wusrc[HC:2 * HC, :].astype(jnp.bfloat16)
    wd_ref[:, 0:HC] = wdsrc[:, 0:HC].astype(jnp.bfloat16)
    wd_ref[:, HC:2 * HC] = wdsrc[:, HC:2 * HC].astype(jnp.bfloat16)

    lane = jax.lax.broadcasted_iota(jnp.int32, (TB, E), 1)
    dn = (((1,), (1,)), ((), ()))
    for tb in range(NTB):
        rows = pl.ds(tb * TB, TB)
        xb = xbf_ref[rows, :]
        hb = tb % 2
        if shared:
            w = 1.0
        else:
            w = jnp.sum(jnp.where(lane == e, wfull_ref[rows, :], 0.0),
                        axis=-1, keepdims=True)
        g0 = jax.lax.dot_general(xb, wgu_ref[0 * HC:1 * HC, :], dn,
                                 preferred_element_type=jnp.float32)
        u0 = jax.lax.dot_general(xb, wgu_ref[1 * HC:2 * HC, :], dn,
                                 preferred_element_type=jnp.float32)
        hs_ref[hb, :, 0:HC] = ((g0 * (1.0 / (1.0 + jnp.exp(-g0)))) * u0 * w
                               ).astype(jnp.bfloat16)
        g1 = jax.lax.dot_general(xb, wgu_ref[2 * HC:3 * HC, :], dn,
                                 preferred_element_type=jnp.float32)
        u1 = jax.lax.dot_general(xb, wgu_ref[3 * HC:4 * HC, :], dn,
                                 preferred_element_type=jnp.float32)
        hs_ref[hb, :, HC:2 * HC] = ((g1 * (1.0 / (1.0 + jnp.exp(-g1)))) * u1 * w
                                    ).astype(jnp.bfloat16)
        o = jax.lax.dot_general(hs_ref[hb], wd_ref[...], dn,
                                preferred_element_type=jnp.float32)
        out_ref[rows, :] += o


def _moe_body(x_ref, gate_ref, w1g_ref, w1u_ref, w2_ref, swg_ref, swu_ref,
              swd_ref, out_ref, xbf_ref, wfull_ref, wgu_ref, wd_ref, hs_ref):
    e = pl.program_id(0)

    @pl.when(e == 0)
    def _router():
        x = x_ref[...]
        xbf_ref[...] = x.astype(jnp.bfloat16)
        logits = jax.lax.dot_general(
            x, gate_ref[...], (((1,), (1,)), ((), ())),
            preferred_element_type=jnp.float32)
        m = jnp.max(logits, axis=-1, keepdims=True)
        ex = jnp.exp(logits - m)
        probs = ex / jnp.sum(ex, axis=-1, keepdims=True)
        lane = jax.lax.broadcasted_iota(jnp.int32, (T, E), 1)
        v1 = jnp.max(probs, axis=-1, keepdims=True)
        i1 = jnp.min(jnp.where(probs == v1, lane, E), axis=-1, keepdims=True)
        m1 = lane == i1
        probs2 = jnp.where(m1, -1.0, probs)
        v2 = jnp.max(probs2, axis=-1, keepdims=True)
        i2 = jnp.min(jnp.where(probs2 == v2, lane, E), axis=-1, keepdims=True)
        m2 = lane == i2
        denom = v1 + v2
        wfull_ref[...] = (jnp.where(m1, v1, 0.0) + jnp.where(m2, v2, 0.0)) / denom
        out_ref[...] = jnp.zeros((T, D), jnp.float32)

    @pl.when(e < E)
    def _routed():
        _expert_pass(w1g_ref[0], w1u_ref[0], w2_ref[0], e, xbf_ref,
                     wfull_ref, out_ref, wgu_ref, wd_ref, hs_ref, shared=False)

    @pl.when(e == E)
    def _shared():
        _expert_pass(swg_ref[...], swu_ref[...], swd_ref[...], e, xbf_ref,
                     wfull_ref, out_ref, wgu_ref, wd_ref, hs_ref, shared=True)


@jax.jit
def kernel(hidden_states, gate_w, w1_gate, w1_up, w2, sw_gate, sw_up, sw_down):
    grid = (E + 1,)
    out = pl.pallas_call(
        _moe_body,
        grid=grid,
        in_specs=[
            pl.BlockSpec((T, D), lambda e: (0, 0)),
            pl.BlockSpec((E, D), lambda e: (0, 0)),
            pl.BlockSpec((1, DFF, D), lambda e: (jnp.minimum(e, E - 1), 0, 0)),
            pl.BlockSpec((1, DFF, D), lambda e: (jnp.minimum(e, E - 1), 0, 0)),
            pl.BlockSpec((1, D, DFF), lambda e: (jnp.minimum(e, E - 1), 0, 0)),
            pl.BlockSpec((DFF, D), lambda e: (0, 0)),
            pl.BlockSpec((DFF, D), lambda e: (0, 0)),
            pl.BlockSpec((D, DFF), lambda e: (0, 0)),
        ],
        out_specs=pl.BlockSpec((T, D), lambda e: (0, 0)),
        out_shape=jax.ShapeDtypeStruct((T, D), jnp.float32),
        scratch_shapes=[
            pltpu.VMEM((T, D), jnp.bfloat16),
            pltpu.VMEM((T, E), jnp.float32),
            pltpu.VMEM((2 * DFF, D), jnp.bfloat16),
            pltpu.VMEM((D, DFF), jnp.bfloat16),
            pltpu.VMEM((2, TB, DFF), jnp.bfloat16),
        ],
        compiler_params=pltpu.CompilerParams(
            dimension_semantics=("arbitrary",)),
    )(hidden_states, gate_w, w1_gate, w1_up, w2, sw_gate, sw_up, sw_down)
    return out
